# baseline (speedup 1.0000x reference)
import jax
import jax.numpy as jnp
from jax import lax
from jax.experimental import pallas as pl
from jax.experimental.pallas import tpu as pltpu
from jax.experimental.pallas import tpu_sc as plsc

V = 4096
B, L = 1024, 20
N = B * L
NC, NS = 2, 16
NW = NC * NS
RPW = N // NW
K = 8
STEPS = RPW // K
LANES = 16


def _sc_gather_body(table_hbm, idx_hbm, tgt_hbm, out_hbm, tpart_hbm,
                    idx_v, tgt_v, acc_v,
                    rows0, rows1, sg0, sg1, so0, so1):
    wid = lax.axis_index("s") * NC + lax.axis_index("c")
    base = wid * RPW
    bufs = (rows0, rows1)
    gsems = (sg0, sg1)
    osems = (so0, so1)

    pltpu.sync_copy(idx_hbm.at[pl.ds(base, RPW)], idx_v)
    pltpu.sync_copy(tgt_hbm.at[pl.ds(base, RPW + LANES - K)], tgt_v)

    def start_gather(g, b):
        src = table_hbm.at[idx_v.at[pl.ds(g * K, K)]]
        pltpu.make_async_copy(src, bufs[b], gsems[b]).start()

    def wait_gather(b):
        pltpu.make_async_copy(
            table_hbm.at[idx_v.at[pl.ds(0, K)]], bufs[b], gsems[b]
        ).wait()

    def start_out(g, b):
        dst = out_hbm.at[pl.ds(base + g * K, K)]
        pltpu.make_async_copy(bufs[b], dst, osems[b]).start()

    def wait_out(b):
        pltpu.make_async_copy(
            bufs[b], out_hbm.at[pl.ds(base, K)], osems[b]
        ).wait()

    start_gather(0, 0)
    lane = lax.iota(jnp.int32, LANES)
    valid = lane < K
    row_sel = jnp.where(valid, lane, 0)

    def step(i, acc):
        for b in range(2):
            g = 2 * i + b
            ob = 1 - b

            @pl.when(g >= 1)
            def _():
                wait_out(ob)

            @pl.when(g + 1 < STEPS)
            def _():
                start_gather(g + 1, ob)

            wait_gather(b)
            tv = tgt_v[pl.ds(g * K, LANES)]
            got = plsc.load_gather(bufs[b], [row_sel, tv], mask=valid)
            acc = acc + jnp.where(valid, got, 0.0)
            start_out(g, b)
        return acc

    acc = lax.fori_loop(
        0, STEPS // 2, step, jnp.zeros((LANES,), jnp.float32)
    )
    wait_out(1)
    acc_v[...] = acc
    pltpu.sync_copy(acc_v, tpart_hbm.at[pl.ds(wid * LANES, LANES)])


_sc_gather = pl.kernel(
    _sc_gather_body,
    out_type=(
        jax.ShapeDtypeStruct((N, V), jnp.float32),
        jax.ShapeDtypeStruct((NW * LANES,), jnp.float32),
    ),
    mesh=plsc.VectorSubcoreMesh(core_axis_name="c", subcore_axis_name="s"),
    compiler_params=pltpu.CompilerParams(needs_layout_passes=False),
    scratch_types=[
        pltpu.VMEM((RPW,), jnp.int32),
        pltpu.VMEM((RPW + LANES - K,), jnp.int32),
        pltpu.VMEM((LANES,), jnp.float32),
        pltpu.VMEM((K, V), jnp.float32),
        pltpu.VMEM((K, V), jnp.float32),
        pltpu.SemaphoreType.DMA,
        pltpu.SemaphoreType.DMA,
        pltpu.SemaphoreType.DMA,
        pltpu.SemaphoreType.DMA,
    ],
)


def _sc_lse_gather_body(lse_hbm, idx_hbm, part_hbm, lse_v, idx_v, acc_v):
    wid = lax.axis_index("s") * NC + lax.axis_index("c")
    base = wid * RPW
    pltpu.sync_copy(lse_hbm, lse_v)
    pltpu.sync_copy(idx_hbm.at[pl.ds(base, RPW)], idx_v)

    acc = jnp.zeros((LANES,), jnp.float32)
    for j in range(RPW // LANES):
        iv = idx_v[pl.ds(j * LANES, LANES)]
        acc = acc + plsc.load_gather(lse_v, [iv])
    acc_v[...] = acc
    pltpu.sync_copy(acc_v, part_hbm.at[pl.ds(wid * LANES, LANES)])


_sc_lse_gather = pl.kernel(
    _sc_lse_gather_body,
    out_type=jax.ShapeDtypeStruct((NW * LANES,), jnp.float32),
    mesh=plsc.VectorSubcoreMesh(core_axis_name="c", subcore_axis_name="s"),
    compiler_params=pltpu.CompilerParams(needs_layout_passes=False),
    scratch_types=[
        pltpu.VMEM((V,), jnp.float32),
        pltpu.VMEM((RPW,), jnp.int32),
        pltpu.VMEM((LANES,), jnp.float32),
    ],
)


LSE_BLK = 256
LSE_BLKS = V // LSE_BLK


def _tc_lse_body(table_ref, lse_ref):
    x = table_ref[...]
    m = jnp.max(x, axis=1)
    s = jnp.sum(jnp.exp(x - m[:, None]), axis=1)
    lse_ref[...] = jnp.log(s) + m


_tc_lse = pl.pallas_call(
    _tc_lse_body,
    grid=(LSE_BLKS,),
    in_specs=[pl.BlockSpec((LSE_BLK, V), lambda i: (i, 0))],
    out_specs=pl.BlockSpec((LSE_BLK,), lambda i: (i,)),
    out_shape=jax.ShapeDtypeStruct((V,), jnp.float32),
)


def _tc_combine_body(lsep_ref, tgtp_ref, out_ref):
    out_ref[0, 0] = (jnp.sum(lsep_ref[...]) - jnp.sum(tgtp_ref[...])) / N


_tc_combine = pl.pallas_call(
    _tc_combine_body,
    out_specs=pl.BlockSpec(memory_space=pltpu.SMEM),
    out_shape=jax.ShapeDtypeStruct((1, 1), jnp.float32),
)


def _tc_retile_body(flat_ref, out_ref):
    out_ref[...] = flat_ref[...].reshape(1, L, V)


_tc_retile = pl.pallas_call(
    _tc_retile_body,
    grid=(B,),
    in_specs=[pl.BlockSpec((L * V,), lambda b: (b,))],
    out_specs=pl.BlockSpec((1, L, V), lambda b: (b, 0, 0)),
    out_shape=jax.ShapeDtypeStruct((B, L, V), jnp.float32),
)


def kernel(indices, targets, table):
    idx = indices.reshape(-1).astype(jnp.int32)
    tgt = jnp.pad(targets.reshape(-1).astype(jnp.int32), (0, LANES - K))
    logits_flat, tgt_parts = _sc_gather(table, idx, tgt)
    lse = _tc_lse(table)
    lse_parts = _sc_lse_gather(lse, idx)
    loss = _tc_combine(lse_parts, tgt_parts)
    logits = _tc_retile(logits_flat.reshape(-1))
    return logits, loss[0, 0]


# speedup vs baseline: 1.8233x; 1.8233x over previous
import jax
import jax.numpy as jnp
from jax import lax
from jax.experimental import pallas as pl
from jax.experimental.pallas import tpu as pltpu
from jax.experimental.pallas import tpu_sc as plsc

V = 4096
B, L = 1024, 20
N = B * L
NC, NS = 2, 16
NW = NC * NS
RPW = N // NW
K = 8
STEPS = RPW // K
LANES = 16


def _sc_gather_body(table_hbm, idx_hbm, tgt_hbm, out_hbm, tpart_hbm,
                    idx_v, tgt_v, acc_v,
                    rows0, rows1, rows2, sg0, sg1, sg2, so0, so1, so2):
    wid = lax.axis_index("s") * NC + lax.axis_index("c")
    base = wid * RPW
    bufs = (rows0, rows1, rows2)
    gsems = (sg0, sg1, sg2)
    osems = (so0, so1, so2)

    pltpu.sync_copy(idx_hbm.at[pl.ds(base, RPW)], idx_v)
    pltpu.sync_copy(tgt_hbm.at[pl.ds(base, RPW + LANES - K)], tgt_v)

    def start_gather(g, b):
        src = table_hbm.at[idx_v.at[pl.ds(g * K, K)]]
        pltpu.make_async_copy(src, bufs[b], gsems[b]).start()

    def wait_gather(b):
        pltpu.make_async_copy(
            table_hbm.at[idx_v.at[pl.ds(0, K)]], bufs[b], gsems[b]
        ).wait()

    def start_out(g, b):
        dst = out_hbm.at[pl.ds(base + g * K, K)]
        pltpu.make_async_copy(bufs[b], dst, osems[b]).start()

    def wait_out(b):
        pltpu.make_async_copy(
            bufs[b], out_hbm.at[pl.ds(base, K)], osems[b]
        ).wait()

    start_gather(0, 0)
    start_gather(1, 1)
    lane = lax.iota(jnp.int32, LANES)
    valid = lane < K
    row_sel = jnp.where(valid, lane, 0)

    def extract(g, b, acc):
        tv = tgt_v[pl.ds(g * K, LANES)]
        got = plsc.load_gather(bufs[b], [row_sel, tv], mask=valid)
        return acc + jnp.where(valid, got, 0.0)

    def step(i, acc):
        for b in range(3):
            g = 3 * i + b

            @pl.when(g >= 1)
            def _():
                wait_out((b + 2) % 3)

            @pl.when(g + 2 < STEPS)
            def _():
                start_gather(g + 2, (b + 2) % 3)

            wait_gather(b)
            acc = extract(g, b, acc)
            start_out(g, b)
        return acc

    acc = lax.fori_loop(
        0, STEPS // 3, step, jnp.zeros((LANES,), jnp.float32)
    )
    for g in range(STEPS - STEPS % 3, STEPS):
        b = g % 3
        wait_out((b + 2) % 3)
        wait_gather(b)
        acc = extract(g, b, acc)
        start_out(g, b)
    wait_out((STEPS - 1) % 3)
    acc_v[...] = acc
    pltpu.sync_copy(acc_v, tpart_hbm.at[pl.ds(wid * LANES, LANES)])


_sc_gather = pl.kernel(
    _sc_gather_body,
    out_type=(
        jax.ShapeDtypeStruct((N, V), jnp.float32),
        jax.ShapeDtypeStruct((NW * LANES,), jnp.float32),
    ),
    mesh=plsc.VectorSubcoreMesh(core_axis_name="c", subcore_axis_name="s"),
    compiler_params=pltpu.CompilerParams(needs_layout_passes=False),
    scratch_types=[
        pltpu.VMEM((RPW,), jnp.int32),
        pltpu.VMEM((RPW + LANES - K,), jnp.int32),
        pltpu.VMEM((LANES,), jnp.float32),
        pltpu.VMEM((K, V), jnp.float32),
        pltpu.VMEM((K, V), jnp.float32),
        pltpu.VMEM((K, V), jnp.float32),
        pltpu.SemaphoreType.DMA,
        pltpu.SemaphoreType.DMA,
        pltpu.SemaphoreType.DMA,
        pltpu.SemaphoreType.DMA,
        pltpu.SemaphoreType.DMA,
        pltpu.SemaphoreType.DMA,
    ],
)


def _sc_lse_gather_body(lse_hbm, idx_hbm, part_hbm, lse_v, idx_v, acc_v):
    wid = lax.axis_index("s") * NC + lax.axis_index("c")
    base = wid * RPW
    pltpu.sync_copy(lse_hbm, lse_v)
    pltpu.sync_copy(idx_hbm.at[pl.ds(base, RPW)], idx_v)

    acc = jnp.zeros((LANES,), jnp.float32)
    for j in range(RPW // LANES):
        iv = idx_v[pl.ds(j * LANES, LANES)]
        acc = acc + plsc.load_gather(lse_v, [iv])
    acc_v[...] = acc
    pltpu.sync_copy(acc_v, part_hbm.at[pl.ds(wid * LANES, LANES)])


_sc_lse_gather = pl.kernel(
    _sc_lse_gather_body,
    out_type=jax.ShapeDtypeStruct((NW * LANES,), jnp.float32),
    mesh=plsc.VectorSubcoreMesh(core_axis_name="c", subcore_axis_name="s"),
    compiler_params=pltpu.CompilerParams(needs_layout_passes=False),
    scratch_types=[
        pltpu.VMEM((V,), jnp.float32),
        pltpu.VMEM((RPW,), jnp.int32),
        pltpu.VMEM((LANES,), jnp.float32),
    ],
)


LSE_BLK = 256
LSE_BLKS = V // LSE_BLK


def _tc_lse_body(table_ref, lse_ref):
    x = table_ref[...]
    m = jnp.max(x, axis=1)
    s = jnp.sum(jnp.exp(x - m[:, None]), axis=1)
    lse_ref[...] = jnp.log(s) + m


_tc_lse = pl.pallas_call(
    _tc_lse_body,
    grid=(LSE_BLKS,),
    in_specs=[pl.BlockSpec((LSE_BLK, V), lambda i: (i, 0))],
    out_specs=pl.BlockSpec((LSE_BLK,), lambda i: (i,)),
    out_shape=jax.ShapeDtypeStruct((V,), jnp.float32),
)


def _tc_combine_body(lsep_ref, tgtp_ref, out_ref):
    out_ref[0, 0] = (jnp.sum(lsep_ref[...]) - jnp.sum(tgtp_ref[...])) / N


_tc_combine = pl.pallas_call(
    _tc_combine_body,
    out_specs=pl.BlockSpec(memory_space=pltpu.SMEM),
    out_shape=jax.ShapeDtypeStruct((1, 1), jnp.float32),
)


def kernel(indices, targets, table):
    idx = indices.reshape(-1).astype(jnp.int32)
    tgt = jnp.pad(targets.reshape(-1).astype(jnp.int32), (0, LANES - K))
    logits_flat, tgt_parts = _sc_gather(table, idx, tgt)
    lse = _tc_lse(table)
    lse_parts = _sc_lse_gather(lse, idx)
    loss = _tc_combine(lse_parts, tgt_parts)
    return logits_flat.reshape(indices.shape + (V,)), loss[0, 0]


# speedup vs baseline: 1.8237x; 1.0002x over previous
import jax
import jax.numpy as jnp
from jax import lax
from jax.experimental import pallas as pl
from jax.experimental.pallas import tpu as pltpu
from jax.experimental.pallas import tpu_sc as plsc

V = 4096
B, L = 1024, 20
N = B * L
NC, NS = 2, 16
NW = NC * NS
RPW = N // NW
K = 8
STEPS = RPW // K
LANES = 16


def _sc_gather_body(table_hbm, idx_hbm, tgt_hbm, out_hbm, tpart_hbm,
                    idx_v, tgt_v, acc_v,
                    rows0, rows1, rows2, sg0, sg1, sg2, so0, so1, so2):
    wid = lax.axis_index("s") * NC + lax.axis_index("c")
    base = wid * RPW
    bufs = (rows0, rows1, rows2)
    gsems = (sg0, sg1, sg2)
    osems = (so0, so1, so2)

    pltpu.sync_copy(idx_hbm.at[pl.ds(base, RPW)], idx_v)
    pltpu.sync_copy(tgt_hbm.at[pl.ds(base, RPW + LANES - K)], tgt_v)

    def start_gather(g, b):
        src = table_hbm.at[idx_v.at[pl.ds(g * K, K)]]
        pltpu.make_async_copy(src, bufs[b], gsems[b]).start()

    def wait_gather(b):
        pltpu.make_async_copy(
            table_hbm.at[idx_v.at[pl.ds(0, K)]], bufs[b], gsems[b]
        ).wait()

    def start_out(g, b):
        dst = out_hbm.at[pl.ds(base + g * K, K)]
        pltpu.make_async_copy(bufs[b], dst, osems[b]).start()

    def wait_out(b):
        pltpu.make_async_copy(
            bufs[b], out_hbm.at[pl.ds(base, K)], osems[b]
        ).wait()

    start_gather(0, 0)
    start_gather(1, 1)
    lane = lax.iota(jnp.int32, LANES)
    valid = lane < K
    row_sel = jnp.where(valid, lane, 0)

    def extract(g, b, acc):
        tv = tgt_v[pl.ds(g * K, LANES)]
        got = plsc.load_gather(bufs[b], [row_sel, tv], mask=valid)
        return acc + jnp.where(valid, got, 0.0)

    def step(i, acc):
        for b in range(3):
            g = 3 * i + b

            @pl.when(g >= 1)
            def _():
                wait_out((b + 2) % 3)

            @pl.when(g + 2 < STEPS)
            def _():
                start_gather(g + 2, (b + 2) % 3)

            wait_gather(b)
            start_out(g, b)
            acc = extract(g, b, acc)
        return acc

    acc = lax.fori_loop(
        0, STEPS // 3, step, jnp.zeros((LANES,), jnp.float32)
    )
    for g in range(STEPS - STEPS % 3, STEPS):
        b = g % 3
        wait_out((b + 2) % 3)
        wait_gather(b)
        start_out(g, b)
        acc = extract(g, b, acc)
    wait_out((STEPS - 1) % 3)
    acc_v[...] = acc
    pltpu.sync_copy(acc_v, tpart_hbm.at[pl.ds(wid * LANES, LANES)])


_sc_gather = pl.kernel(
    _sc_gather_body,
    out_type=(
        jax.ShapeDtypeStruct((N, V), jnp.float32),
        jax.ShapeDtypeStruct((NW * LANES,), jnp.float32),
    ),
    mesh=plsc.VectorSubcoreMesh(core_axis_name="c", subcore_axis_name="s"),
    compiler_params=pltpu.CompilerParams(needs_layout_passes=False),
    scratch_types=[
        pltpu.VMEM((RPW,), jnp.int32),
        pltpu.VMEM((RPW + LANES - K,), jnp.int32),
        pltpu.VMEM((LANES,), jnp.float32),
        pltpu.VMEM((K, V), jnp.float32),
        pltpu.VMEM((K, V), jnp.float32),
        pltpu.VMEM((K, V), jnp.float32),
        pltpu.SemaphoreType.DMA,
        pltpu.SemaphoreType.DMA,
        pltpu.SemaphoreType.DMA,
        pltpu.SemaphoreType.DMA,
        pltpu.SemaphoreType.DMA,
        pltpu.SemaphoreType.DMA,
    ],
)


def _sc_lse_gather_body(lse_hbm, idx_hbm, part_hbm, lse_v, idx_v, acc_v):
    wid = lax.axis_index("s") * NC + lax.axis_index("c")
    base = wid * RPW
    pltpu.sync_copy(lse_hbm, lse_v)
    pltpu.sync_copy(idx_hbm.at[pl.ds(base, RPW)], idx_v)

    acc = jnp.zeros((LANES,), jnp.float32)
    for j in range(RPW // LANES):
        iv = idx_v[pl.ds(j * LANES, LANES)]
        acc = acc + plsc.load_gather(lse_v, [iv])
    acc_v[...] = acc
    pltpu.sync_copy(acc_v, part_hbm.at[pl.ds(wid * LANES, LANES)])


_sc_lse_gather = pl.kernel(
    _sc_lse_gather_body,
    out_type=jax.ShapeDtypeStruct((NW * LANES,), jnp.float32),
    mesh=plsc.VectorSubcoreMesh(core_axis_name="c", subcore_axis_name="s"),
    compiler_params=pltpu.CompilerParams(needs_layout_passes=False),
    scratch_types=[
        pltpu.VMEM((V,), jnp.float32),
        pltpu.VMEM((RPW,), jnp.int32),
        pltpu.VMEM((LANES,), jnp.float32),
    ],
)


LSE_BLK = 256
LSE_BLKS = V // LSE_BLK


def _tc_lse_body(table_ref, lse_ref):
    x = table_ref[...]
    m = jnp.max(x, axis=1)
    s = jnp.sum(jnp.exp(x - m[:, None]), axis=1)
    lse_ref[...] = jnp.log(s) + m


_tc_lse = pl.pallas_call(
    _tc_lse_body,
    grid=(LSE_BLKS,),
    in_specs=[pl.BlockSpec((LSE_BLK, V), lambda i: (i, 0))],
    out_specs=pl.BlockSpec((LSE_BLK,), lambda i: (i,)),
    out_shape=jax.ShapeDtypeStruct((V,), jnp.float32),
)


def _tc_combine_body(lsep_ref, tgtp_ref, out_ref):
    out_ref[0, 0] = (jnp.sum(lsep_ref[...]) - jnp.sum(tgtp_ref[...])) / N


_tc_combine = pl.pallas_call(
    _tc_combine_body,
    out_specs=pl.BlockSpec(memory_space=pltpu.SMEM),
    out_shape=jax.ShapeDtypeStruct((1, 1), jnp.float32),
)


def kernel(indices, targets, table):
    idx = indices.reshape(-1).astype(jnp.int32)
    tgt = jnp.pad(targets.reshape(-1).astype(jnp.int32), (0, LANES - K))
    logits_flat, tgt_parts = _sc_gather(table, idx, tgt)
    lse = _tc_lse(table)
    lse_parts = _sc_lse_gather(lse, idx)
    loss = _tc_combine(lse_parts, tgt_parts)
    return logits_flat.reshape(indices.shape + (V,)), loss[0, 0]
